# baseline (speedup 1.0000x reference)
import functools

import jax
import jax.numpy as jnp
from jax import lax
from jax.experimental import pallas as pl
from jax.experimental.pallas import tpu as pltpu
from jax.experimental.pallas import tpu_sc as plsc

DIM = 64
S = 8
K = 4


@functools.cache
def _make_gather(b: int, l: int, dim: int):
    info = plsc.get_sparse_core_info()
    nw = info.num_cores * info.num_subcores
    assert b % nw == 0 and b // nw % S == 0
    rows_per_w = b // nw
    l_a = min(l, 128)
    l_b = l - l_a
    mesh = plsc.VectorSubcoreMesh(core_axis_name="c", subcore_axis_name="s")

    @functools.partial(
        pl.kernel,
        mesh=mesh,
        out_type=jax.ShapeDtypeStruct((b, l, dim), jnp.float32),
        scratch_types=[
            pltpu.VMEM((rows_per_w, l_a), jnp.int32),
            pltpu.VMEM((rows_per_w, l_b), jnp.int32),
            pltpu.VMEM((S, l, dim), jnp.float32),
            pltpu.SemaphoreType.DMA,
            pltpu.SemaphoreType.DMA,
        ],
        compiler_params=pltpu.CompilerParams(use_tc_tiling_on_sc=False),
    )
    def gather_kernel(tok_hbm, table_hbm, out_hbm, idx_a, idx_b, rows_v, gsem, osem):
        wid = lax.axis_index("s") * info.num_cores + lax.axis_index("c")
        base = wid * rows_per_w
        pltpu.sync_copy(tok_hbm.at[pl.ds(base, rows_per_w), pl.ds(0, l_a)], idx_a)
        pltpu.sync_copy(tok_hbm.at[pl.ds(base, rows_per_w), pl.ds(l_a, l_b)], idx_b)

        def issue_gather(j, slot):
            pltpu.async_copy(
                table_hbm.at[idx_a.at[j]], rows_v.at[slot, pl.ds(0, l_a)], gsem
            )
            pltpu.async_copy(
                table_hbm.at[idx_b.at[j]], rows_v.at[slot, pl.ds(l_a, l_b)], gsem
            )

        def wait_gather(j, slot):
            pltpu.make_async_copy(
                table_hbm.at[idx_a.at[j]], rows_v.at[slot, pl.ds(0, l_a)], gsem
            ).wait()
            pltpu.make_async_copy(
                table_hbm.at[idx_b.at[j]], rows_v.at[slot, pl.ds(l_a, l_b)], gsem
            ).wait()

        for c in range(K):
            issue_gather(c, c % S)

        @pl.loop(0, rows_per_w, step=S)
        def outer(j0):
            for s in range(S):
                j = j0 + s
                jk = j + K
                sk = (s + K) % S

                @pl.when(jk < rows_per_w)
                def _issue():
                    @pl.when(jk >= S)
                    def _drain():
                        pltpu.make_async_copy(
                            rows_v.at[sk], out_hbm.at[base], osem
                        ).wait()

                    issue_gather(jk, sk)

                wait_gather(j, s)
                pltpu.async_copy(rows_v.at[s], out_hbm.at[base + j], osem)

        for _ in range(S):
            pltpu.make_async_copy(rows_v.at[0], out_hbm.at[base], osem).wait()

    return gather_kernel


def kernel(tok, table):
    b, l = tok.shape
    gather_kernel = _make_gather(b, l, DIM)
    return gather_kernel(tok, table)


# speedup vs baseline: 1.2260x; 1.2260x over previous
import functools

import jax
import jax.numpy as jnp
from jax import lax
from jax.experimental import pallas as pl
from jax.experimental.pallas import tpu as pltpu
from jax.experimental.pallas import tpu_sc as plsc

DIM = 64
WIDE = 128
CHUNK = 40
S = 8
K = 4


@functools.cache
def _make_gather(b: int, l: int, dim: int):
    info = plsc.get_sparse_core_info()
    nw = info.num_cores * info.num_subcores
    assert b % nw == 0
    rows_per_w = b // nw
    assert l % CHUNK == 0
    chunks_per_row = l // CHUNK
    n_chunks = rows_per_w * chunks_per_row
    assert n_chunks % S == 0
    mesh = plsc.VectorSubcoreMesh(core_axis_name="c", subcore_axis_name="s")

    @functools.partial(
        pl.kernel,
        mesh=mesh,
        out_type=jax.ShapeDtypeStruct((b, l, WIDE), jnp.float32),
        scratch_types=[
            pltpu.VMEM((rows_per_w * l,), jnp.int32),
            pltpu.VMEM((S, CHUNK, WIDE), jnp.float32),
            pltpu.SemaphoreType.DMA,
            pltpu.SemaphoreType.DMA,
        ],
        compiler_params=pltpu.CompilerParams(use_tc_tiling_on_sc=True),
    )
    def gather_kernel(tok_hbm, table_hbm, out_hbm, idx_v, rows_v, gsem, osem):
        wid = lax.axis_index("s") * info.num_cores + lax.axis_index("c")
        base = wid * rows_per_w
        pltpu.sync_copy(tok_hbm.at[pl.ds(base * l, rows_per_w * l)], idx_v)

        def issue_gather(c, slot):
            pltpu.async_copy(
                table_hbm.at[idx_v.at[pl.ds(c * CHUNK, CHUNK)]],
                rows_v.at[slot],
                gsem,
            )

        def wait_gather(slot):
            pltpu.make_async_copy(
                table_hbm.at[idx_v.at[pl.ds(0, CHUNK)]],
                rows_v.at[slot],
                gsem,
            ).wait()

        def issue_write(c, slot):
            b_sub = c // chunks_per_row
            m = c % chunks_per_row
            pltpu.async_copy(
                rows_v.at[slot],
                out_hbm.at[base + b_sub, pl.ds(m * CHUNK, CHUNK)],
                osem,
            )

        def wait_write(slot):
            pltpu.make_async_copy(
                rows_v.at[slot],
                out_hbm.at[base, pl.ds(0, CHUNK)],
                osem,
            ).wait()

        for c in range(K):
            issue_gather(c, c % S)

        @pl.loop(0, n_chunks, step=S)
        def outer(c0):
            for s in range(S):
                c = c0 + s
                ck = c + K
                sk = (s + K) % S

                @pl.when(ck < n_chunks)
                def _issue():
                    @pl.when(ck >= S)
                    def _drain():
                        wait_write(sk)

                    issue_gather(ck, sk)

                wait_gather(s)
                issue_write(c, s)

        for s in range(S):
            wait_write(s)

    return gather_kernel


def kernel(tok, table):
    b, l = tok.shape
    gather_kernel = _make_gather(b, l, DIM)
    table_wide = jnp.pad(table, ((0, 0), (0, WIDE - DIM)))
    out_wide = gather_kernel(tok.reshape(-1), table_wide)
    return out_wide[:, :, :DIM]
